# baseline (speedup 1.0000x reference)
import jax
import jax.numpy as jnp
from jax import lax
from jax.experimental import pallas as pl
from jax.experimental.pallas import tpu as pltpu
from jax.experimental.pallas import tpu_sc as plsc

N_CLS = 20
_LANES = 16
_CHUNK = 2048
_G = _CHUNK // _LANES
_NW = 32


def _sc_body(x_ref, out_ref, buf_a, buf_b, lab_a, lab_b, sem_a, sem_b):
    hw_total = out_ref.shape[2]
    per_w = (out_ref.shape[0] * hw_total) // _NW
    n_chunks = per_w // _CHUNK
    w_per_img = hw_total // per_w

    cid = lax.axis_index("c")
    sid = lax.axis_index("s")
    wid = sid * 2 + cid
    base = wid * per_w
    b = wid // w_per_img
    hw0 = (wid % w_per_img) * per_w

    iota = lax.iota(jnp.int32, _LANES)
    ones = jnp.full((_LANES,), 1.0, jnp.float32)
    zeros = jnp.zeros((_LANES,), jnp.float32)

    for buf in (buf_a, buf_b):
        for r in range(N_CLS):
            def zrow(i, _, buf=buf, r=r):
                buf[r, pl.ds(i * _LANES, _LANES)] = zeros
                return 0
            lax.fori_loop(0, _G, zrow, 0)

    def do_chunk(k, buf, labv, sem, first):
        off = k * _CHUNK
        dst = out_ref.at[b, :, pl.ds(hw0 + off, _CHUNK)]

        @pl.when(jnp.logical_not(first))
        def _():
            pltpu.make_async_copy(buf, dst, sem).wait()

            def unscatter(g, _):
                goff = g * _LANES
                lab = labv[pl.ds(goff, _LANES)]
                plsc.store_scatter(buf, [lab, goff + iota], zeros)
                return 0
            lax.fori_loop(0, _G, unscatter, 0)

        pltpu.sync_copy(x_ref.at[pl.ds(base + off, _CHUNK)], labv)

        def scatter(g, _):
            goff = g * _LANES
            lab = labv[pl.ds(goff, _LANES)]
            plsc.store_scatter(buf, [lab, goff + iota], ones)
            return 0
        lax.fori_loop(0, _G, scatter, 0)

        pltpu.make_async_copy(buf, dst, sem).start()

    def pair(kk, _):
        do_chunk(2 * kk, buf_a, lab_a, sem_a, kk == 0)
        do_chunk(2 * kk + 1, buf_b, lab_b, sem_b, kk == 0)
        return 0
    lax.fori_loop(0, n_chunks // 2, pair, 0)

    tail = hw0 + (n_chunks - 2) * _CHUNK
    pltpu.make_async_copy(
        buf_a, out_ref.at[b, :, pl.ds(tail, _CHUNK)], sem_a).wait()
    pltpu.make_async_copy(
        buf_b, out_ref.at[b, :, pl.ds(tail + _CHUNK, _CHUNK)], sem_b).wait()


def kernel(x):
    B, _, H, W = x.shape
    HW = H * W
    xf = x.reshape(B * HW)
    mesh = plsc.VectorSubcoreMesh(core_axis_name="c", subcore_axis_name="s")
    f = pl.kernel(
        _sc_body,
        out_type=jax.ShapeDtypeStruct((B, N_CLS, HW), jnp.float32),
        mesh=mesh,
        compiler_params=pltpu.CompilerParams(
            use_tc_tiling_on_sc=False, needs_layout_passes=False),
        scratch_types=[
            pltpu.VMEM((N_CLS, _CHUNK), jnp.float32),
            pltpu.VMEM((N_CLS, _CHUNK), jnp.float32),
            pltpu.VMEM((_CHUNK,), jnp.int32),
            pltpu.VMEM((_CHUNK,), jnp.int32),
            pltpu.SemaphoreType.DMA,
            pltpu.SemaphoreType.DMA,
        ],
    )
    out = f(xf)
    return out.reshape(B, N_CLS, H, W)


# speedup vs baseline: 1.1901x; 1.1901x over previous
import jax
import jax.numpy as jnp
from jax import lax
from jax.experimental import pallas as pl
from jax.experimental.pallas import tpu as pltpu
from jax.experimental.pallas import tpu_sc as plsc

N_CLS = 20
_LANES = 16
_CHUNK = 2048
_G = _CHUNK // _LANES
_NW = 32


def _sc_body(x_ref, out_ref,
             buf_a, buf_b, lab0, lab1, lab2, lab3,
             sem_a, sem_b, lsem0, lsem1):
    hw_total = out_ref.shape[2]
    per_w = (out_ref.shape[0] * hw_total) // _NW
    n_chunks = per_w // _CHUNK
    w_per_img = hw_total // per_w

    cid = lax.axis_index("c")
    sid = lax.axis_index("s")
    wid = sid * 2 + cid
    base = wid * per_w
    b = wid // w_per_img
    hw0 = (wid % w_per_img) * per_w

    iota = lax.iota(jnp.int32, _LANES)
    ones = jnp.full((_LANES,), 1.0, jnp.float32)
    zeros = jnp.zeros((_LANES,), jnp.float32)
    labs = (lab0, lab1, lab2, lab3)
    lsems = (lsem0, lsem1)

    for buf in (buf_a, buf_b):
        for r in range(N_CLS):
            def zrow(i, _, buf=buf, r=r):
                buf[r, pl.ds(i * _LANES, _LANES)] = zeros
                return 0
            lax.fori_loop(0, _G, zrow, 0)

    def lab_start(k, labv, lsem):
        pltpu.make_async_copy(
            x_ref.at[pl.ds(base + k * _CHUNK, _CHUNK)], labv, lsem).start()

    def lab_wait(k, labv, lsem):
        pltpu.make_async_copy(
            x_ref.at[pl.ds(base + k * _CHUNK, _CHUNK)], labv, lsem).wait()

    lab_start(0, labs[0], lsems[0])
    lab_start(1, labs[1], lsems[1])

    def do_chunk(k, buf, sem, lab_new, lab_old, lsem_new, first):
        off = k * _CHUNK
        dst = out_ref.at[b, :, pl.ds(hw0 + off, _CHUNK)]

        def drain_and_unscatter():
            pltpu.make_async_copy(buf, dst, sem).wait()

            @plsc.parallel_loop(0, _G, unroll=8)
            def _unscatter(g):
                goff = g * _LANES
                lab = lab_old[pl.ds(goff, _LANES)]
                plsc.store_scatter(buf, [lab, goff + iota], zeros)

        if first is None:
            drain_and_unscatter()
        else:
            pl.when(jnp.logical_not(first))(drain_and_unscatter)

        lab_wait(k, lab_new, lsem_new)

        @plsc.parallel_loop(0, _G, unroll=8)
        def _scatter(g):
            goff = g * _LANES
            lab = lab_new[pl.ds(goff, _LANES)]
            plsc.store_scatter(buf, [lab, goff + iota], ones)

        pltpu.make_async_copy(buf, dst, sem).start()

    def quad(qq, _):
        k0 = 4 * qq
        first = qq == 0
        do_chunk(k0 + 0, buf_a, sem_a, labs[0], labs[2], lsems[0], first)
        lab_start(k0 + 2, labs[2], lsems[0])
        do_chunk(k0 + 1, buf_b, sem_b, labs[1], labs[3], lsems[1], first)
        lab_start(k0 + 3, labs[3], lsems[1])
        do_chunk(k0 + 2, buf_a, sem_a, labs[2], labs[0], lsems[0], None)

        @pl.when(qq < (n_chunks // 4) - 1)
        def _():
            lab_start(k0 + 4, labs[0], lsems[0])
        do_chunk(k0 + 3, buf_b, sem_b, labs[3], labs[1], lsems[1], None)

        @pl.when(qq < (n_chunks // 4) - 1)
        def _():
            lab_start(k0 + 5, labs[1], lsems[1])
        return 0

    lax.fori_loop(0, n_chunks // 4, quad, 0)

    tail = hw0 + (n_chunks - 2) * _CHUNK
    pltpu.make_async_copy(
        buf_a, out_ref.at[b, :, pl.ds(tail, _CHUNK)], sem_a).wait()
    pltpu.make_async_copy(
        buf_b, out_ref.at[b, :, pl.ds(tail + _CHUNK, _CHUNK)], sem_b).wait()


def kernel(x):
    B, _, H, W = x.shape
    HW = H * W
    xf = x.reshape(B * HW)
    mesh = plsc.VectorSubcoreMesh(core_axis_name="c", subcore_axis_name="s")
    f = pl.kernel(
        _sc_body,
        out_type=jax.ShapeDtypeStruct((B, N_CLS, HW), jnp.float32),
        mesh=mesh,
        compiler_params=pltpu.CompilerParams(
            use_tc_tiling_on_sc=False, needs_layout_passes=False),
        scratch_types=[
            pltpu.VMEM((N_CLS, _CHUNK), jnp.float32),
            pltpu.VMEM((N_CLS, _CHUNK), jnp.float32),
            pltpu.VMEM((_CHUNK,), jnp.int32),
            pltpu.VMEM((_CHUNK,), jnp.int32),
            pltpu.VMEM((_CHUNK,), jnp.int32),
            pltpu.VMEM((_CHUNK,), jnp.int32),
            pltpu.SemaphoreType.DMA,
            pltpu.SemaphoreType.DMA,
            pltpu.SemaphoreType.DMA,
            pltpu.SemaphoreType.DMA,
        ],
    )
    out = f(xf)
    return out.reshape(B, N_CLS, H, W)


# speedup vs baseline: 1.2203x; 1.0254x over previous
import jax
import jax.numpy as jnp
from jax import lax
from jax.experimental import pallas as pl
from jax.experimental.pallas import tpu as pltpu
from jax.experimental.pallas import tpu_sc as plsc

N_CLS = 20
_LANES = 16
_CHUNK = 2048
_G = _CHUNK // _LANES
_NW = 32


def _sc_body(x_ref, out_ref,
             buf_a, buf_b, lab0, lab1, lab2, lab3,
             sem_a, sem_b, lsem0, lsem1):
    B, _, H, W = x_ref.shape
    hw_total = H * W
    per_w = (B * hw_total) // _NW
    n_chunks = per_w // _CHUNK
    w_per_img = hw_total // per_w
    crows = _CHUNK // W
    gpr = W // _LANES

    cid = lax.axis_index("c")
    sid = lax.axis_index("s")
    wid = sid * 2 + cid
    b = wid // w_per_img
    row0 = (wid % w_per_img) * (per_w // W)

    iota = lax.iota(jnp.int32, _LANES)
    ones = jnp.full((_LANES,), 1.0, jnp.float32)
    zeros = jnp.zeros((_LANES,), jnp.float32)
    labs = (lab0, lab1, lab2, lab3)
    lsems = (lsem0, lsem1)

    for buf in (buf_a, buf_b):
        for c in range(N_CLS):
            for r in range(crows):
                def zrow(i, _, buf=buf, c=c, r=r):
                    buf[c, r, pl.ds(i * _LANES, _LANES)] = zeros
                    return 0
                lax.fori_loop(0, gpr, zrow, 0)

    def lab_start(k, labv, lsem):
        pltpu.make_async_copy(
            x_ref.at[b, 0, pl.ds(row0 + k * crows, crows), :],
            labv, lsem).start()

    def lab_wait(k, labv, lsem):
        pltpu.make_async_copy(
            x_ref.at[b, 0, pl.ds(row0 + k * crows, crows), :],
            labv, lsem).wait()

    lab_start(0, labs[0], lsems[0])
    lab_start(1, labs[1], lsems[1])

    def do_chunk(k, buf, sem, lab_new, lab_old, lsem_new, first):
        dst = out_ref.at[b, :, pl.ds(row0 + k * crows, crows), :]

        def drain_and_unscatter():
            pltpu.make_async_copy(buf, dst, sem).wait()

            @plsc.parallel_loop(0, _G, unroll=8)
            def _unscatter(g):
                r = g // gpr
                coff = (g % gpr) * _LANES
                lab = lab_old[r, pl.ds(coff, _LANES)]
                plsc.store_scatter(
                    buf, [lab, jnp.full((_LANES,), r, jnp.int32),
                          coff + iota], zeros)

        if first is None:
            drain_and_unscatter()
        else:
            pl.when(jnp.logical_not(first))(drain_and_unscatter)

        lab_wait(k, lab_new, lsem_new)

        @plsc.parallel_loop(0, _G, unroll=8)
        def _scatter(g):
            r = g // gpr
            coff = (g % gpr) * _LANES
            lab = lab_new[r, pl.ds(coff, _LANES)]
            plsc.store_scatter(
                buf, [lab, jnp.full((_LANES,), r, jnp.int32),
                      coff + iota], ones)

        pltpu.make_async_copy(buf, dst, sem).start()

    def quad(qq, _):
        k0 = 4 * qq
        first = qq == 0
        do_chunk(k0 + 0, buf_a, sem_a, labs[0], labs[2], lsems[0], first)
        lab_start(k0 + 2, labs[2], lsems[0])
        do_chunk(k0 + 1, buf_b, sem_b, labs[1], labs[3], lsems[1], first)
        lab_start(k0 + 3, labs[3], lsems[1])
        do_chunk(k0 + 2, buf_a, sem_a, labs[2], labs[0], lsems[0], None)

        @pl.when(qq < (n_chunks // 4) - 1)
        def _():
            lab_start(k0 + 4, labs[0], lsems[0])
        do_chunk(k0 + 3, buf_b, sem_b, labs[3], labs[1], lsems[1], None)

        @pl.when(qq < (n_chunks // 4) - 1)
        def _():
            lab_start(k0 + 5, labs[1], lsems[1])
        return 0

    lax.fori_loop(0, n_chunks // 4, quad, 0)

    tr = row0 + (n_chunks - 2) * crows
    pltpu.make_async_copy(
        buf_a, out_ref.at[b, :, pl.ds(tr, crows), :], sem_a).wait()
    pltpu.make_async_copy(
        buf_b, out_ref.at[b, :, pl.ds(tr + crows, crows), :], sem_b).wait()


def kernel(x):
    B, _, H, W = x.shape
    crows = _CHUNK // W
    mesh = plsc.VectorSubcoreMesh(core_axis_name="c", subcore_axis_name="s")
    f = pl.kernel(
        _sc_body,
        out_type=jax.ShapeDtypeStruct((B, N_CLS, H, W), jnp.float32),
        mesh=mesh,
        compiler_params=pltpu.CompilerParams(
            use_tc_tiling_on_sc=False, needs_layout_passes=False),
        scratch_types=[
            pltpu.VMEM((N_CLS, crows, W), jnp.float32),
            pltpu.VMEM((N_CLS, crows, W), jnp.float32),
            pltpu.VMEM((crows, W), jnp.int32),
            pltpu.VMEM((crows, W), jnp.int32),
            pltpu.VMEM((crows, W), jnp.int32),
            pltpu.VMEM((crows, W), jnp.int32),
            pltpu.SemaphoreType.DMA,
            pltpu.SemaphoreType.DMA,
            pltpu.SemaphoreType.DMA,
            pltpu.SemaphoreType.DMA,
        ],
    )
    return f(x)


# speedup vs baseline: 3.8933x; 3.1904x over previous
import jax
import jax.numpy as jnp
from jax import lax
from jax.experimental import pallas as pl
from jax.experimental.pallas import tpu as pltpu
from jax.experimental.pallas import tpu_sc as plsc

N_CLS = 20
_LANES = 16
_CROWS = 8
_CCOLS = 256
_CHUNK = _CROWS * _CCOLS
_G = _CHUNK // _LANES
_GPR = _CCOLS // _LANES
_NW = 32


def _sc_body(x_ref, out_ref,
             buf_a, buf_b, lab0, lab1, lab2, lab3,
             sem_a, sem_b, lsem0, lsem1):
    B, _, H, W = x_ref.shape
    per_w = (B * H * W) // _NW
    n_chunks = per_w // _CHUNK
    w_per_img = (H * W) // per_w
    cchunks = W // _CCOLS

    cid = lax.axis_index("c")
    sid = lax.axis_index("s")
    wid = sid * 2 + cid
    b = wid // w_per_img
    row0 = (wid % w_per_img) * (per_w // W)

    iota = lax.iota(jnp.int32, _LANES)
    ones = jnp.full((_LANES,), 1.0, jnp.float32)
    zeros = jnp.zeros((_LANES,), jnp.float32)
    labs = (lab0, lab1, lab2, lab3)
    lsems = (lsem0, lsem1)

    for buf in (buf_a, buf_b):
        for c in range(N_CLS):
            for r in range(_CROWS):
                def zrow(i, _, buf=buf, c=c, r=r):
                    buf[c, r, pl.ds(i * _LANES, _LANES)] = zeros
                    return 0
                lax.fori_loop(0, _GPR, zrow, 0)

    def _slices(k):
        rc = k // cchunks
        cc = k % cchunks
        return pl.ds(row0 + rc * _CROWS, _CROWS), pl.ds(cc * _CCOLS, _CCOLS)

    def lab_start(k, labv, lsem):
        rs, cs = _slices(k)
        pltpu.make_async_copy(x_ref.at[b, 0, rs, cs], labv, lsem).start()

    def lab_wait(k, labv, lsem):
        rs, cs = _slices(k)
        pltpu.make_async_copy(x_ref.at[b, 0, rs, cs], labv, lsem).wait()

    lab_start(0, labs[0], lsems[0])
    lab_start(1, labs[1], lsems[1])

    def do_chunk(k, buf, sem, lab_new, lab_old, lsem_new, first):
        rs, cs = _slices(k)
        dst = out_ref.at[b, :, rs, cs]

        def drain_and_unscatter():
            pltpu.make_async_copy(buf, dst, sem).wait()

            @plsc.parallel_loop(0, _G, unroll=8)
            def _unscatter(g):
                r = g // _GPR
                coff = (g % _GPR) * _LANES
                lab = lab_old[r, pl.ds(coff, _LANES)]
                plsc.store_scatter(
                    buf, [lab, jnp.full((_LANES,), r, jnp.int32),
                          coff + iota], zeros)

        if first is None:
            drain_and_unscatter()
        else:
            pl.when(jnp.logical_not(first))(drain_and_unscatter)

        lab_wait(k, lab_new, lsem_new)

        @plsc.parallel_loop(0, _G, unroll=8)
        def _scatter(g):
            r = g // _GPR
            coff = (g % _GPR) * _LANES
            lab = lab_new[r, pl.ds(coff, _LANES)]
            plsc.store_scatter(
                buf, [lab, jnp.full((_LANES,), r, jnp.int32),
                      coff + iota], ones)

        pltpu.make_async_copy(buf, dst, sem).start()

    def quad(qq, _):
        k0 = 4 * qq
        first = qq == 0
        do_chunk(k0 + 0, buf_a, sem_a, labs[0], labs[2], lsems[0], first)
        lab_start(k0 + 2, labs[2], lsems[0])
        do_chunk(k0 + 1, buf_b, sem_b, labs[1], labs[3], lsems[1], first)
        lab_start(k0 + 3, labs[3], lsems[1])
        do_chunk(k0 + 2, buf_a, sem_a, labs[2], labs[0], lsems[0], None)

        @pl.when(qq < (n_chunks // 4) - 1)
        def _():
            lab_start(k0 + 4, labs[0], lsems[0])
        do_chunk(k0 + 3, buf_b, sem_b, labs[3], labs[1], lsems[1], None)

        @pl.when(qq < (n_chunks // 4) - 1)
        def _():
            lab_start(k0 + 5, labs[1], lsems[1])
        return 0

    lax.fori_loop(0, n_chunks // 4, quad, 0)

    rs_a, cs_a = _slices(n_chunks - 2)
    rs_b, cs_b = _slices(n_chunks - 1)
    pltpu.make_async_copy(buf_a, out_ref.at[b, :, rs_a, cs_a], sem_a).wait()
    pltpu.make_async_copy(buf_b, out_ref.at[b, :, rs_b, cs_b], sem_b).wait()


def kernel(x):
    B, _, H, W = x.shape
    mesh = plsc.VectorSubcoreMesh(core_axis_name="c", subcore_axis_name="s")
    f = pl.kernel(
        _sc_body,
        out_type=jax.ShapeDtypeStruct((B, N_CLS, H, W), jnp.float32),
        mesh=mesh,
        compiler_params=pltpu.CompilerParams(
            use_tc_tiling_on_sc=True, needs_layout_passes=False),
        scratch_types=[
            pltpu.VMEM((N_CLS, _CROWS, _CCOLS), jnp.float32),
            pltpu.VMEM((N_CLS, _CROWS, _CCOLS), jnp.float32),
            pltpu.VMEM((_CROWS, _CCOLS), jnp.int32),
            pltpu.VMEM((_CROWS, _CCOLS), jnp.int32),
            pltpu.VMEM((_CROWS, _CCOLS), jnp.int32),
            pltpu.VMEM((_CROWS, _CCOLS), jnp.int32),
            pltpu.SemaphoreType.DMA,
            pltpu.SemaphoreType.DMA,
            pltpu.SemaphoreType.DMA,
            pltpu.SemaphoreType.DMA,
        ],
    )
    return f(x)


# speedup vs baseline: 4.0244x; 1.0337x over previous
import jax
import jax.numpy as jnp
from jax import lax
from jax.experimental import pallas as pl
from jax.experimental.pallas import tpu as pltpu
from jax.experimental.pallas import tpu_sc as plsc

N_CLS = 20
_LANES = 16
_CROWS = 8
_CCOLS = 256
_CHUNK = _CROWS * _CCOLS
_G = _CHUNK // _LANES
_GPR = _CCOLS // _LANES
_NW = 32


def _sc_body(x_ref, out_ref,
             buf_a, buf_b, lab0, lab1, lab2, lab3,
             sem_a, sem_b, lsem0, lsem1):
    B, _, H, W = x_ref.shape
    per_w = (B * H * W) // _NW
    n_chunks = per_w // _CHUNK
    w_per_img = (H * W) // per_w
    cchunks = W // _CCOLS

    cid = lax.axis_index("c")
    sid = lax.axis_index("s")
    wid = sid * 2 + cid
    b = wid // w_per_img
    row0 = (wid % w_per_img) * (per_w // W)

    iota = lax.iota(jnp.int32, _LANES)
    ones = jnp.full((_LANES,), 1.0, jnp.float32)
    zeros = jnp.zeros((_LANES,), jnp.float32)
    labs = (lab0, lab1, lab2, lab3)
    lsems = (lsem0, lsem1)

    for buf in (buf_a, buf_b):
        for c in range(N_CLS):
            for r in range(_CROWS):
                def zrow(i, _, buf=buf, c=c, r=r):
                    buf[c, r, pl.ds(i * _LANES, _LANES)] = zeros
                    return 0
                lax.fori_loop(0, _GPR, zrow, 0)

    def _slices(k):
        rc = k // cchunks
        cc = k % cchunks
        return pl.ds(row0 + rc * _CROWS, _CROWS), pl.ds(cc * _CCOLS, _CCOLS)

    def lab_start(k, labv, lsem):
        rs, cs = _slices(k)
        pltpu.make_async_copy(x_ref.at[b, 0, rs, cs], labv, lsem).start()

    def lab_wait(k, labv, lsem):
        rs, cs = _slices(k)
        pltpu.make_async_copy(x_ref.at[b, 0, rs, cs], labv, lsem).wait()

    lab_start(0, labs[0], lsems[0])
    lab_start(1, labs[1], lsems[1])

    def out_start(k, buf, sem):
        rs, cs = _slices(k)
        for c in range(N_CLS):
            pltpu.make_async_copy(
                buf.at[c], out_ref.at[b, c, rs, cs], sem).start()

    def out_wait(k, buf, sem):
        rs, cs = _slices(k)
        for c in range(N_CLS):
            pltpu.make_async_copy(
                buf.at[c], out_ref.at[b, c, rs, cs], sem).wait()

    def do_chunk(k, buf, sem, lab_new, lab_old, lsem_new, first):
        def drain_and_unscatter():
            out_wait(k, buf, sem)

            @plsc.parallel_loop(0, _G, unroll=8)
            def _unscatter(g):
                r = g // _GPR
                coff = (g % _GPR) * _LANES
                lab = lab_old[r, pl.ds(coff, _LANES)]
                plsc.store_scatter(
                    buf, [lab, jnp.full((_LANES,), r, jnp.int32),
                          coff + iota], zeros)

        if first is None:
            drain_and_unscatter()
        else:
            pl.when(jnp.logical_not(first))(drain_and_unscatter)

        lab_wait(k, lab_new, lsem_new)

        @plsc.parallel_loop(0, _G, unroll=8)
        def _scatter(g):
            r = g // _GPR
            coff = (g % _GPR) * _LANES
            lab = lab_new[r, pl.ds(coff, _LANES)]
            plsc.store_scatter(
                buf, [lab, jnp.full((_LANES,), r, jnp.int32),
                      coff + iota], ones)

        out_start(k, buf, sem)

    def quad(qq, _):
        k0 = 4 * qq
        first = qq == 0
        do_chunk(k0 + 0, buf_a, sem_a, labs[0], labs[2], lsems[0], first)
        lab_start(k0 + 2, labs[2], lsems[0])
        do_chunk(k0 + 1, buf_b, sem_b, labs[1], labs[3], lsems[1], first)
        lab_start(k0 + 3, labs[3], lsems[1])
        do_chunk(k0 + 2, buf_a, sem_a, labs[2], labs[0], lsems[0], None)

        @pl.when(qq < (n_chunks // 4) - 1)
        def _():
            lab_start(k0 + 4, labs[0], lsems[0])
        do_chunk(k0 + 3, buf_b, sem_b, labs[3], labs[1], lsems[1], None)

        @pl.when(qq < (n_chunks // 4) - 1)
        def _():
            lab_start(k0 + 5, labs[1], lsems[1])
        return 0

    lax.fori_loop(0, n_chunks // 4, quad, 0)

    out_wait(n_chunks - 2, buf_a, sem_a)
    out_wait(n_chunks - 1, buf_b, sem_b)


def kernel(x):
    B, _, H, W = x.shape
    mesh = plsc.VectorSubcoreMesh(core_axis_name="c", subcore_axis_name="s")
    f = pl.kernel(
        _sc_body,
        out_type=jax.ShapeDtypeStruct((B, N_CLS, H, W), jnp.float32),
        mesh=mesh,
        compiler_params=pltpu.CompilerParams(
            use_tc_tiling_on_sc=True, needs_layout_passes=False),
        scratch_types=[
            pltpu.VMEM((N_CLS, _CROWS, _CCOLS), jnp.float32),
            pltpu.VMEM((N_CLS, _CROWS, _CCOLS), jnp.float32),
            pltpu.VMEM((_CROWS, _CCOLS), jnp.int32),
            pltpu.VMEM((_CROWS, _CCOLS), jnp.int32),
            pltpu.VMEM((_CROWS, _CCOLS), jnp.int32),
            pltpu.VMEM((_CROWS, _CCOLS), jnp.int32),
            pltpu.SemaphoreType.DMA,
            pltpu.SemaphoreType.DMA,
            pltpu.SemaphoreType.DMA,
            pltpu.SemaphoreType.DMA,
        ],
    )
    return f(x)
